# baseline (speedup 1.0000x reference)
import functools

import jax
import jax.numpy as jnp
from jax import lax
from jax.experimental import pallas as pl
from jax.experimental.pallas import tpu as pltpu
from jax.experimental.pallas import tpu_sc as plsc

T, D, E, F = 2048, 768, 8, 1536
K = 2
NP = T * K
B = 128
P = NP + E * B
NBLK = P // B
NW = 32
PPW = NP // NW
TPW = T // NW


def _erf(x):
    s = jnp.sign(x)
    a = jnp.abs(x)
    t = 1.0 / (1.0 + 0.3275911 * a)
    poly = t * (0.254829592 + t * (-0.284496736 + t * (1.421413741
              + t * (-1.453152027 + t * 1.061405429))))
    return s * (1.0 - poly * jnp.exp(-a * a))


def _gelu_exact(x):
    return 0.5 * x * (1.0 + _erf(x * 0.7071067811865476))


def _router_body(x_ref, rw_ref, rb_ref,
                 dest_ref, scale_ref, be_ref, bv_ref,
                 m_ref, r_ref):
    logits = jnp.dot(x_ref[...], rw_ref[...],
                     preferred_element_type=jnp.float32) + rb_ref[...]
    p = jax.nn.softmax(logits, axis=-1)
    e_iota = lax.broadcasted_iota(jnp.int32, p.shape, 1)
    m1 = jnp.max(p, axis=-1, keepdims=True)
    i1 = jnp.argmax(p, axis=-1)[:, None]
    p2 = jnp.where(e_iota == i1, -jnp.inf, p)
    m2 = jnp.max(p2, axis=-1, keepdims=True)
    i2 = jnp.argmax(p2, axis=-1)[:, None]
    w0 = 1.0 / (1.0 + jnp.exp(m2 - m1))
    scale_ref[pl.ds(0, T), :] = w0
    scale_ref[pl.ds(T, T), :] = 1.0 - w0

    m_ref[pl.ds(0, T), :] = (e_iota == i1).astype(jnp.float32)
    m_ref[pl.ds(T, T), :] = (e_iota == i2).astype(jnp.float32)

    c_iota = lax.broadcasted_iota(jnp.int32, (B, B), 1)
    r_iota = lax.broadcasted_iota(jnp.int32, (B, B), 0)
    tril_excl = (r_iota > c_iota).astype(jnp.float32)

    def chunk(c, running):
        mc = m_ref[pl.ds(c * B, B), :]
        r_ref[pl.ds(c * B, B), :] = running + jnp.dot(
            tril_excl, mc, preferred_element_type=jnp.float32)
        return running + jnp.sum(mc, axis=0, keepdims=True)

    counts = lax.fori_loop(0, NP // B, chunk, jnp.zeros((1, E), jnp.float32))

    ci = counts.astype(jnp.int32)
    pc = ((ci + B - 1) // B) * B
    e_r = lax.broadcasted_iota(jnp.int32, (E, E), 0)
    e_c = lax.broadcasted_iota(jnp.int32, (E, E), 1)
    triu_strict = (e_r < e_c).astype(jnp.float32)
    pcf = pc.astype(jnp.float32)
    poff = jnp.dot(pcf, triu_strict, preferred_element_type=jnp.float32)
    cum_incl = poff + pcf

    m = m_ref[...]
    dest = jnp.sum(m * (r_ref[...] + poff), axis=1, keepdims=True)
    dest_ref[...] = dest.astype(jnp.int32)

    bb = (lax.broadcasted_iota(jnp.int32, (NBLK, E), 0) * B).astype(jnp.float32)
    be = jnp.sum((bb >= cum_incl).astype(jnp.int32), axis=1, keepdims=True)
    be_ref[...] = jnp.minimum(be, E - 1)
    total = jnp.sum(pcf)
    bv_ref[...] = (bb[:, 0:1] < total).astype(jnp.int32)


def _router(x, router_w, rb):
    return pl.pallas_call(
        _router_body,
        in_specs=[
            pl.BlockSpec((T, D), lambda: (0, 0)),
            pl.BlockSpec((D, E), lambda: (0, 0)),
            pl.BlockSpec((1, E), lambda: (0, 0)),
        ],
        out_specs=[
            pl.BlockSpec((NP, 1), lambda: (0, 0)),
            pl.BlockSpec((NP, 1), lambda: (0, 0)),
            pl.BlockSpec((NBLK, 1), lambda: (0, 0)),
            pl.BlockSpec((NBLK, 1), lambda: (0, 0)),
        ],
        out_shape=[
            jax.ShapeDtypeStruct((NP, 1), jnp.int32),
            jax.ShapeDtypeStruct((NP, 1), jnp.float32),
            jax.ShapeDtypeStruct((NBLK, 1), jnp.int32),
            jax.ShapeDtypeStruct((NBLK, 1), jnp.int32),
        ],
        scratch_shapes=[
            pltpu.VMEM((NP, E), jnp.float32),
            pltpu.VMEM((NP, E), jnp.float32),
        ],
    )(x, router_w, rb)


@functools.cache
def _sc_mesh():
    return plsc.VectorSubcoreMesh(core_axis_name="c", subcore_axis_name="s")


@functools.cache
def _dispatch_kernel():
    @functools.partial(
        pl.kernel,
        out_type=(jax.ShapeDtypeStruct((P, D), jnp.float32),
                  jax.ShapeDtypeStruct((P,), jnp.float32)),
        mesh=_sc_mesh(),
        scratch_types=[
            pltpu.VMEM((PPW,), jnp.int32),
            pltpu.VMEM((PPW, D), jnp.float32),
            pltpu.VMEM((PPW,), jnp.float32),
            pltpu.SemaphoreType.DMA,
            pltpu.SemaphoreType.DMA,
        ],
    )
    def _dispatch(x_hbm, dest_hbm, scale_hbm, xg_hbm, ss_hbm,
                  dest_v, rows_v, scale_v, sem1, sem2):
        wid = lax.axis_index("s") * 2 + lax.axis_index("c")
        base = wid * PPW
        tok_base = lax.rem(base, T)
        pltpu.sync_copy(dest_hbm.at[pl.ds(base, PPW)], dest_v)
        pltpu.sync_copy(scale_hbm.at[pl.ds(base, PPW)], scale_v)
        pltpu.sync_copy(x_hbm.at[pl.ds(tok_base, PPW)], rows_v)
        cp1 = pltpu.async_copy(rows_v, xg_hbm.at[dest_v], sem1)
        cp2 = pltpu.async_copy(scale_v, ss_hbm.at[dest_v], sem2)
        cp1.wait()
        cp2.wait()

    return _dispatch


def _experts_body(be_ref, bv_ref, xg_ref, w1_ref, b1_ref, w2_ref, b2_ref,
                  ss_ref, yg_ref):
    b = pl.program_id(0)

    @pl.when(bv_ref[b] == 1)
    def _():
        h = jnp.dot(xg_ref[...], w1_ref[0],
                    preferred_element_type=jnp.float32) + b1_ref[0]
        h = _gelu_exact(h)
        y = jnp.dot(h, w2_ref[0], preferred_element_type=jnp.float32) + b2_ref[0]
        yg_ref[...] = y * ss_ref[...]


def _experts(be, bv, xg, w1, b1r, w2, b2r, ss2):
    grid_spec = pltpu.PrefetchScalarGridSpec(
        num_scalar_prefetch=2,
        grid=(NBLK,),
        in_specs=[
            pl.BlockSpec((B, D), lambda b, be, bv: (b, 0)),
            pl.BlockSpec((1, D, F), lambda b, be, bv: (be[b], 0, 0)),
            pl.BlockSpec((1, 1, F), lambda b, be, bv: (be[b], 0, 0)),
            pl.BlockSpec((1, F, D), lambda b, be, bv: (be[b], 0, 0)),
            pl.BlockSpec((1, 1, D), lambda b, be, bv: (be[b], 0, 0)),
            pl.BlockSpec((B, 1), lambda b, be, bv: (b, 0)),
        ],
        out_specs=pl.BlockSpec((B, D), lambda b, be, bv: (b, 0)),
        scratch_shapes=[],
    )
    return pl.pallas_call(
        _experts_body,
        grid_spec=grid_spec,
        out_shape=jax.ShapeDtypeStruct((P, D), jnp.float32),
    )(be, bv, xg, w1, b1r, w2, b2r, ss2)


@functools.cache
def _combine_kernel():
    @functools.partial(
        pl.kernel,
        out_type=jax.ShapeDtypeStruct((T, D), jnp.float32),
        mesh=_sc_mesh(),
        scratch_types=[
            pltpu.VMEM((TPW,), jnp.int32),
            pltpu.VMEM((TPW,), jnp.int32),
            pltpu.VMEM((TPW, D), jnp.float32),
            pltpu.VMEM((TPW, D), jnp.float32),
            pltpu.SemaphoreType.DMA,
            pltpu.SemaphoreType.DMA,
        ],
    )
    def _combine(yg_hbm, dest_hbm, out_hbm,
                 d0_v, d1_v, rows_a, rows_b, sem_a, sem_b):
        wid = lax.axis_index("s") * 2 + lax.axis_index("c")
        base = wid * TPW
        pltpu.sync_copy(dest_hbm.at[pl.ds(base, TPW)], d0_v)
        pltpu.sync_copy(dest_hbm.at[pl.ds(T + base, TPW)], d1_v)
        cp_a = pltpu.async_copy(yg_hbm.at[d0_v], rows_a, sem_a)
        cp_b = pltpu.async_copy(yg_hbm.at[d1_v], rows_b, sem_b)
        cp_a.wait()
        cp_b.wait()

        def row_add(r, carry):
            for cc in range(D // 16):
                sl = pl.ds(cc * 16, 16)
                rows_a[r, sl] = rows_a[r, sl] + rows_b[r, sl]
            return carry

        lax.fori_loop(0, TPW, row_add, 0)
        pltpu.sync_copy(rows_a, out_hbm.at[pl.ds(base, TPW)])

    return _combine


@jax.jit
def kernel(x, router_w, router_b, w1, b1, w2, b2):
    rb = router_b.reshape(1, E)
    b1r = b1.reshape(E, 1, F)
    b2r = b2.reshape(E, 1, D)
    dest2, scale2, be2, bv2 = _router(x, router_w, rb)
    dest = dest2.reshape(NP)
    scale = scale2.reshape(NP)
    be = be2.reshape(NBLK)
    bv = bv2.reshape(NBLK)
    xg, ss = _dispatch_kernel()(x, dest, scale)
    yg = _experts(be, bv, xg, w1, b1r, w2, b2r, ss.reshape(P, 1))
    return _combine_kernel()(yg, dest)
